# baseline (speedup 1.0000x reference)
import functools

import jax
import jax.numpy as jnp
from jax import lax
from jax.experimental import pallas as pl
from jax.experimental.pallas import tpu as pltpu
from jax.experimental.pallas import tpu_sc as plsc

STEP_LEN = 8
MAX_SIZE = 400.0
MIN_SIZE = 1.0
DEFAULT_SIZE = 10.0

NC = 2
NS = 16
L = 16

B = 32
T = 4096
D = 256
ND = D // L
CH = 64
NBUF = 3

TROWS = 256
NT = (STEP_LEN * (int(MAX_SIZE) // 2) + TROWS - 1) // TROWS


def _sc_body(x_hbm, cs_hbm, f_hbm, out_hbm, cs_v, f_v, buf0, buf1, buf2,
             acc_v, part_v, shared, sem0, sem1, sem2):
    c = lax.axis_index("c")
    s = lax.axis_index("s")
    b1 = 2 * s + c
    b2 = 2 * (NS - 1 - s) + c

    pltpu.sync_copy(cs_hbm, cs_v)
    pltpu.sync_copy(f_hbm, f_v)

    zero = jnp.zeros((L,), jnp.float32)
    zeros = tuple(zero for _ in range(2 * ND))

    def read_params(b):
        idx = jnp.full((L,), b, jnp.int32)
        cs_vec = plsc.load_gather(cs_v, [idx])
        f_vec = plsc.load_gather(f_v, [idx])
        ps = jnp.clip(cs_vec + DEFAULT_SIZE, MIN_SIZE, MAX_SIZE)
        trunc_v = ps.astype(jnp.int32)
        frac = ps - trunc_v.astype(jnp.float32)
        nb = (trunc_v + jnp.where(frac > 0.0, 1, 0))[0]
        return ps, trunc_v[0], nb, f_vec[0]

    bufs = (buf0, buf1, buf2)
    sems = (sem0, sem1, sem2)

    def do_range(b, lo, hi, ps, trunc_s):
        for j in range(2 * ND):
            acc_v[pl.ds(j * L, L)] = zero
        nch = (hi - lo + CH - 1) // CH

        def flush(acc):
            for j in range(2 * ND):
                acc_v[pl.ds(j * L, L)] = acc_v[pl.ds(j * L, L)] + acc[j]

        def process(g, mybuf, mysem, next_buf, next_sem):
            pltpu.make_async_copy(
                x_hbm.at[b, pl.ds(0, CH), :], mybuf, mysem).wait()

            @pl.when(g + NBUF - 1 < nch)
            def _prefetch():
                pltpu.async_copy(
                    x_hbm.at[b, pl.ds(lo + (g + NBUF - 1) * CH, CH), :],
                    next_buf, next_sem)

            start = lo + g * CH
            tb_last = (start + CH - 1) // STEP_LEN
            fast = jnp.logical_and(start + CH <= hi, tb_last + 1 <= trunc_s)

            @pl.when(fast)
            def _fast():
                def row_body(r, acc):
                    new_m = []
                    new_s = []
                    for j in range(ND):
                        v = mybuf[r, pl.ds(j * L, L)]
                        new_m.append(acc[j] + v)
                        new_s.append(acc[ND + j] + v * v)
                    return tuple(new_m + new_s)

                flush(lax.fori_loop(0, CH, row_body, zeros, unroll=2))

            @pl.when(jnp.logical_not(fast))
            def _slow():
                def row_body(r, acc):
                    t = start + r
                    tb = t // STEP_LEN
                    w = jnp.clip(ps - tb.astype(jnp.float32), 0.0, 1.0)
                    w = w * (t < hi).astype(jnp.float32)
                    new_m = []
                    new_s = []
                    for j in range(ND):
                        v = mybuf[r, pl.ds(j * L, L)]
                        wv = w * v
                        new_m.append(acc[j] + wv)
                        new_s.append(acc[ND + j] + wv * v)
                    return tuple(new_m + new_s)

                flush(lax.fori_loop(0, CH, row_body, zeros, unroll=2))

        for k in range(NBUF - 1):
            @pl.when(k < nch)
            def _prime(k=k):
                pltpu.async_copy(
                    x_hbm.at[b, pl.ds(lo + k * CH, CH), :], bufs[k], sems[k])

        def chunk_body(g, carry):
            for k in range(NBUF):
                @pl.when(g % NBUF == k)
                def _proc(k=k):
                    nk = (k + NBUF - 1) % NBUF
                    process(g, bufs[k], sems[k], bufs[nk], sems[nk])

            return carry

        lax.fori_loop(0, nch, chunk_body, 0, unroll=False)

    ps1, tr1, nb1, f1 = read_params(b1)
    mid1 = f1 + (nb1 - f1 + 1) // 2
    do_range(b1, STEP_LEN * f1, STEP_LEN * mid1, ps1, tr1)
    pltpu.sync_copy(acc_v, shared.at[s, 0])

    ps2, tr2, nb2, f2 = read_params(b2)
    mid2 = f2 + (nb2 - f2 + 1) // 2
    do_range(b2, STEP_LEN * mid2, STEP_LEN * nb2, ps2, tr2)
    pltpu.sync_copy(acc_v, shared.at[NS - 1 - s, 1])

    plsc.subcore_barrier()

    pltpu.sync_copy(shared.at[s, 0], acc_v)
    pltpu.sync_copy(shared.at[s, 1], part_v)
    for j in range(2 * ND):
        part_v[pl.ds(j * L, L)] = (
            acc_v[pl.ds(j * L, L)] + part_v[pl.ds(j * L, L)])
    pltpu.sync_copy(part_v, out_hbm.at[b1])


def _sc_partial(x, current_size, f):
    mesh = plsc.VectorSubcoreMesh(core_axis_name="c", subcore_axis_name="s")
    return pl.kernel(
        _sc_body,
        out_type=jax.ShapeDtypeStruct((B, 2 * D), jnp.float32),
        mesh=mesh,
        compiler_params=pltpu.CompilerParams(needs_layout_passes=False),
        scratch_types=[
            pltpu.VMEM((B,), jnp.float32),
            pltpu.VMEM((B,), jnp.int32),
            pltpu.VMEM((CH, D), jnp.float32),
            pltpu.VMEM((CH, D), jnp.float32),
            pltpu.VMEM((CH, D), jnp.float32),
            pltpu.VMEM((2 * D,), jnp.float32),
            pltpu.VMEM((2 * D,), jnp.float32),
            pltpu.VMEM_SHARED((NS, 2, 2 * D), jnp.float32),
            pltpu.SemaphoreType.DMA,
            pltpu.SemaphoreType.DMA,
            pltpu.SemaphoreType.DMA,
        ],
    )(x, current_size, f)


def _tc_body(tiles_ref, f_ref, psbits_ref, x_ref, out_ref):
    b = pl.program_id(0)
    t = pl.program_id(1)
    psval = lax.bitcast_convert_type(psbits_ref[b], jnp.float32)
    fval = f_ref[b]

    xt = x_ref[0]
    row = t * TROWS + lax.broadcasted_iota(jnp.int32, (TROWS, 1), 0)
    tb = row // STEP_LEN
    w = jnp.clip(psval - tb.astype(jnp.float32), 0.0, 1.0)
    w = jnp.where(tb < fval, w, 0.0)
    wx = xt * w
    m = jnp.sum(wx, axis=0)
    sq = jnp.sum(wx * xt, axis=0)
    contrib = jnp.concatenate([m, sq])[None, None, :]

    @pl.when(t == 0)
    def _init():
        out_ref[...] = contrib

    @pl.when(t > 0)
    def _acc():
        out_ref[...] = out_ref[...] + contrib


def _tc_partial(x, psbits, f, tc_tiles):
    grid_spec = pltpu.PrefetchScalarGridSpec(
        num_scalar_prefetch=3,
        grid=(B, NT),
        in_specs=[
            pl.BlockSpec(
                (1, TROWS, D),
                lambda b, t, tiles, f, pb: (b, jnp.minimum(t, tiles[b] - 1), 0),
            ),
        ],
        out_specs=pl.BlockSpec((1, 1, 2 * D),
                               lambda b, t, tiles, f, pb: (b, 0, 0)),
    )
    return pl.pallas_call(
        _tc_body,
        grid_spec=grid_spec,
        out_shape=jax.ShapeDtypeStruct((B, 1, 2 * D), jnp.float32),
        compiler_params=pltpu.CompilerParams(
            dimension_semantics=("arbitrary", "arbitrary")),
    )(tc_tiles, f, psbits, x)


def _fin_body(tc_ref, sc_ref, inv_ref, out_ref):
    tot = tc_ref[...] + sc_ref[...]
    inv = inv_ref[...]
    mean = tot[:, :D] * inv
    var = tot[:, D:] * inv - mean * mean
    out_ref[...] = jnp.concatenate([mean, var], axis=1)


def _finalize(tc_out, sc_out, inv):
    return pl.pallas_call(
        _fin_body,
        out_shape=jax.ShapeDtypeStruct((B, 2 * D), jnp.float32),
    )(tc_out, sc_out, inv)


@jax.jit
def _run(x, current_size):
    ps = jnp.clip(current_size + DEFAULT_SIZE, MIN_SIZE, MAX_SIZE)
    trunc = ps.astype(jnp.int32)
    frac = ps - trunc.astype(jnp.float32)
    nb = trunc + jnp.where(frac > 0.0, 1, 0)
    f = nb // 2
    tc_tiles = jnp.maximum((f * STEP_LEN + TROWS - 1) // TROWS, 1)
    inv = (1.0 / (float(STEP_LEN) * ps))[:, None]

    sc_out = _sc_partial(x, current_size, f)
    tc_out = _tc_partial(x, lax.bitcast_convert_type(ps, jnp.int32), f,
                         tc_tiles)
    return _finalize(tc_out.reshape(B, 2 * D), sc_out, inv)


def kernel(x, current_size):
    return _run(x, current_size)


# speedup vs baseline: 1.1642x; 1.1642x over previous
import functools

import jax
import jax.numpy as jnp
from jax import lax
from jax.experimental import pallas as pl
from jax.experimental.pallas import tpu as pltpu
from jax.experimental.pallas import tpu_sc as plsc

STEP_LEN = 8
MAX_SIZE = 400.0
MIN_SIZE = 1.0
DEFAULT_SIZE = 10.0

NC = 2
NS = 16
L = 16

B = 32
T = 4096
D = 256
ND = D // L
CH = 64
NBUF = 3

TROWS = 256
NT = (STEP_LEN * (int(MAX_SIZE) // 2) + TROWS - 1) // TROWS


def _sc_body(x_hbm, cs_hbm, f_hbm, out_hbm, cs_v, f_v, buf0, buf1, buf2,
             acc_v, part_v, shared, sem0, sem1, sem2):
    c = lax.axis_index("c")
    s = lax.axis_index("s")
    b1 = 2 * s + c
    b2 = 2 * (NS - 1 - s) + c

    pltpu.sync_copy(cs_hbm, cs_v)
    pltpu.sync_copy(f_hbm, f_v)

    zero = jnp.zeros((L,), jnp.float32)
    zeros = tuple(zero for _ in range(2 * ND))

    def read_params(b):
        idx = jnp.full((L,), b, jnp.int32)
        cs_vec = plsc.load_gather(cs_v, [idx])
        f_vec = plsc.load_gather(f_v, [idx])
        ps = jnp.clip(cs_vec + DEFAULT_SIZE, MIN_SIZE, MAX_SIZE)
        trunc_v = ps.astype(jnp.int32)
        frac = ps - trunc_v.astype(jnp.float32)
        nb = (trunc_v + jnp.where(frac > 0.0, 1, 0))[0]
        return ps, trunc_v[0], nb, f_vec[0]

    bufs = (buf0, buf1, buf2)
    sems = (sem0, sem1, sem2)

    def do_range(b, lo, hi, ps, trunc_s):
        for j in range(2 * ND):
            acc_v[pl.ds(j * L, L)] = zero
        nch = (hi - lo + CH - 1) // CH

        def flush(acc):
            for j in range(2 * ND):
                acc_v[pl.ds(j * L, L)] = acc_v[pl.ds(j * L, L)] + acc[j]

        def process(g, mybuf, mysem, next_buf, next_sem):
            pltpu.make_async_copy(
                x_hbm.at[b, pl.ds(0, CH), :], mybuf, mysem).wait()

            @pl.when(g + NBUF - 1 < nch)
            def _prefetch():
                pltpu.async_copy(
                    x_hbm.at[b, pl.ds(lo + (g + NBUF - 1) * CH, CH), :],
                    next_buf, next_sem)

            start = lo + g * CH
            tb_last = (start + CH - 1) // STEP_LEN
            fast = jnp.logical_and(start + CH <= hi, tb_last + 1 <= trunc_s)

            @pl.when(fast)
            def _fast():
                def row_body(r, acc):
                    new_m = []
                    new_s = []
                    for j in range(ND):
                        v = mybuf[r, pl.ds(j * L, L)]
                        new_m.append(acc[j] + v)
                        new_s.append(acc[ND + j] + v * v)
                    return tuple(new_m + new_s)

                flush(lax.fori_loop(0, CH, row_body, zeros, unroll=2))

            @pl.when(jnp.logical_not(fast))
            def _slow():
                def row_body(r, acc):
                    t = start + r
                    tb = t // STEP_LEN
                    w = jnp.clip(ps - tb.astype(jnp.float32), 0.0, 1.0)
                    w = w * (t < hi).astype(jnp.float32)
                    new_m = []
                    new_s = []
                    for j in range(ND):
                        v = mybuf[r, pl.ds(j * L, L)]
                        wv = w * v
                        new_m.append(acc[j] + wv)
                        new_s.append(acc[ND + j] + wv * v)
                    return tuple(new_m + new_s)

                flush(lax.fori_loop(0, CH, row_body, zeros, unroll=2))

        for k in range(NBUF - 1):
            @pl.when(k < nch)
            def _prime(k=k):
                pltpu.async_copy(
                    x_hbm.at[b, pl.ds(lo + k * CH, CH), :], bufs[k], sems[k])

        def chunk_body(g, carry):
            for k in range(NBUF):
                @pl.when(g % NBUF == k)
                def _proc(k=k):
                    nk = (k + NBUF - 1) % NBUF
                    process(g, bufs[k], sems[k], bufs[nk], sems[nk])

            return carry

        lax.fori_loop(0, nch, chunk_body, 0, unroll=False)

    ps1, tr1, nb1, f1 = read_params(b1)
    mid1 = f1 + (nb1 - f1 + 1) // 2
    do_range(b1, STEP_LEN * f1, STEP_LEN * mid1, ps1, tr1)
    pltpu.sync_copy(acc_v, shared.at[s, 0])

    ps2, tr2, nb2, f2 = read_params(b2)
    mid2 = f2 + (nb2 - f2 + 1) // 2
    do_range(b2, STEP_LEN * mid2, STEP_LEN * nb2, ps2, tr2)
    pltpu.sync_copy(acc_v, shared.at[NS - 1 - s, 1])

    plsc.subcore_barrier()

    pltpu.sync_copy(shared.at[s, 0], acc_v)
    pltpu.sync_copy(shared.at[s, 1], part_v)
    for j in range(2 * ND):
        part_v[pl.ds(j * L, L)] = (
            acc_v[pl.ds(j * L, L)] + part_v[pl.ds(j * L, L)])
    pltpu.sync_copy(part_v, out_hbm.at[b1])


def _sc_partial(x, current_size, f):
    mesh = plsc.VectorSubcoreMesh(core_axis_name="c", subcore_axis_name="s")
    return pl.kernel(
        _sc_body,
        out_type=jax.ShapeDtypeStruct((B, 2 * D), jnp.float32),
        mesh=mesh,
        compiler_params=pltpu.CompilerParams(needs_layout_passes=False),
        scratch_types=[
            pltpu.VMEM((B,), jnp.float32),
            pltpu.VMEM((B,), jnp.int32),
            pltpu.VMEM((CH, D), jnp.float32),
            pltpu.VMEM((CH, D), jnp.float32),
            pltpu.VMEM((CH, D), jnp.float32),
            pltpu.VMEM((2 * D,), jnp.float32),
            pltpu.VMEM((2 * D,), jnp.float32),
            pltpu.VMEM_SHARED((NS, 2, 2 * D), jnp.float32),
            pltpu.SemaphoreType.DMA,
            pltpu.SemaphoreType.DMA,
            pltpu.SemaphoreType.DMA,
        ],
    )(x, current_size, f)


def _tc_body(tiles_ref, f_ref, psbits_ref, x_ref, out_ref):
    b = pl.program_id(0)
    t = pl.program_id(1)
    psval = lax.bitcast_convert_type(psbits_ref[b], jnp.float32)
    fval = f_ref[b]

    @pl.when(t == 0)
    def _init():
        out_ref[...] = jnp.zeros_like(out_ref)

    @pl.when(t * TROWS < fval * STEP_LEN)
    def _acc():
        xt = x_ref[0]
        row = t * TROWS + lax.broadcasted_iota(jnp.int32, (1, TROWS), 1)
        tb = row // STEP_LEN
        w = jnp.clip(psval - tb.astype(jnp.float32), 0.0, 1.0)
        w = jnp.where(tb < fval, w, 0.0)
        m = jax.lax.dot(w, xt, preferred_element_type=jnp.float32)
        sq = jax.lax.dot(w, xt * xt, preferred_element_type=jnp.float32)
        contrib = jnp.concatenate([m, sq], axis=1)[None]
        out_ref[...] = out_ref[...] + contrib


def _tc_partial(x, psbits, f, tc_tiles):
    grid_spec = pltpu.PrefetchScalarGridSpec(
        num_scalar_prefetch=3,
        grid=(B, NT),
        in_specs=[
            pl.BlockSpec(
                (1, TROWS, D),
                lambda b, t, tiles, f, pb: (b, jnp.minimum(t, tiles[b] - 1), 0),
            ),
        ],
        out_specs=pl.BlockSpec((1, 1, 2 * D),
                               lambda b, t, tiles, f, pb: (b, 0, 0)),
    )
    return pl.pallas_call(
        _tc_body,
        grid_spec=grid_spec,
        out_shape=jax.ShapeDtypeStruct((B, 1, 2 * D), jnp.float32),
        compiler_params=pltpu.CompilerParams(
            dimension_semantics=("arbitrary", "arbitrary")),
    )(tc_tiles, f, psbits, x)


def _fin_body(tc_ref, sc_ref, inv_ref, out_ref):
    tot = tc_ref[...] + sc_ref[...]
    inv = inv_ref[...]
    mean = tot[:, :D] * inv
    var = tot[:, D:] * inv - mean * mean
    out_ref[...] = jnp.concatenate([mean, var], axis=1)


def _finalize(tc_out, sc_out, inv):
    return pl.pallas_call(
        _fin_body,
        out_shape=jax.ShapeDtypeStruct((B, 2 * D), jnp.float32),
    )(tc_out, sc_out, inv)


@jax.jit
def _run(x, current_size):
    ps = jnp.clip(current_size + DEFAULT_SIZE, MIN_SIZE, MAX_SIZE)
    trunc = ps.astype(jnp.int32)
    frac = ps - trunc.astype(jnp.float32)
    nb = trunc + jnp.where(frac > 0.0, 1, 0)
    f = nb // 2
    tc_tiles = jnp.maximum((f * STEP_LEN + TROWS - 1) // TROWS, 1)
    inv = (1.0 / (float(STEP_LEN) * ps))[:, None]

    sc_out = _sc_partial(x, current_size, f)
    tc_out = _tc_partial(x, lax.bitcast_convert_type(ps, jnp.int32), f,
                         tc_tiles)
    return _finalize(tc_out.reshape(B, 2 * D), sc_out, inv)


def kernel(x, current_size):
    return _run(x, current_size)


# speedup vs baseline: 1.5037x; 1.2916x over previous
import functools

import jax
import jax.numpy as jnp
from jax import lax
from jax.experimental import pallas as pl
from jax.experimental.pallas import tpu as pltpu
from jax.experimental.pallas import tpu_sc as plsc

STEP_LEN = 8
MAX_SIZE = 400.0
MIN_SIZE = 1.0
DEFAULT_SIZE = 10.0

NC = 2
NS = 16
L = 16

B = 32
T = 4096
D = 256
ND = D // L
CH = 64
NBUF = 3

TROWS = 512
NT = (STEP_LEN * (int(MAX_SIZE) // 2) + TROWS - 1) // TROWS


def _sc_body(x_hbm, cs_hbm, f_hbm, out_hbm, cs_v, f_v, buf0, buf1, buf2,
             acc_v, part_v, shared, sem0, sem1, sem2):
    c = lax.axis_index("c")
    s = lax.axis_index("s")
    b1 = 2 * s + c
    b2 = 2 * (NS - 1 - s) + c

    pltpu.sync_copy(cs_hbm, cs_v)
    pltpu.sync_copy(f_hbm, f_v)

    zero = jnp.zeros((L,), jnp.float32)
    zeros = tuple(zero for _ in range(2 * ND))

    def read_params(b):
        idx = jnp.full((L,), b, jnp.int32)
        cs_vec = plsc.load_gather(cs_v, [idx])
        f_vec = plsc.load_gather(f_v, [idx])
        ps = jnp.clip(cs_vec + DEFAULT_SIZE, MIN_SIZE, MAX_SIZE)
        trunc_v = ps.astype(jnp.int32)
        frac = ps - trunc_v.astype(jnp.float32)
        nb = (trunc_v + jnp.where(frac > 0.0, 1, 0))[0]
        return ps, trunc_v[0], nb, f_vec[0]

    bufs = (buf0, buf1, buf2)
    sems = (sem0, sem1, sem2)

    def do_range(b, lo, hi, ps, trunc_s):
        for j in range(2 * ND):
            acc_v[pl.ds(j * L, L)] = zero
        nch = (hi - lo + CH - 1) // CH

        def flush(acc):
            for j in range(2 * ND):
                acc_v[pl.ds(j * L, L)] = acc_v[pl.ds(j * L, L)] + acc[j]

        def process(g, mybuf, mysem, next_buf, next_sem):
            pltpu.make_async_copy(
                x_hbm.at[b, pl.ds(0, CH), :], mybuf, mysem).wait()

            @pl.when(g + NBUF - 1 < nch)
            def _prefetch():
                pltpu.async_copy(
                    x_hbm.at[b, pl.ds(lo + (g + NBUF - 1) * CH, CH), :],
                    next_buf, next_sem)

            start = lo + g * CH
            tb_last = (start + CH - 1) // STEP_LEN
            fast = jnp.logical_and(start + CH <= hi, tb_last + 1 <= trunc_s)

            @pl.when(fast)
            def _fast():
                def row_body(r, acc):
                    new_m = []
                    new_s = []
                    for j in range(ND):
                        v = mybuf[r, pl.ds(j * L, L)]
                        new_m.append(acc[j] + v)
                        new_s.append(acc[ND + j] + v * v)
                    return tuple(new_m + new_s)

                flush(lax.fori_loop(0, CH, row_body, zeros, unroll=2))

            @pl.when(jnp.logical_not(fast))
            def _slow():
                def row_body(r, acc):
                    t = start + r
                    tb = t // STEP_LEN
                    w = jnp.clip(ps - tb.astype(jnp.float32), 0.0, 1.0)
                    w = w * (t < hi).astype(jnp.float32)
                    new_m = []
                    new_s = []
                    for j in range(ND):
                        v = mybuf[r, pl.ds(j * L, L)]
                        wv = w * v
                        new_m.append(acc[j] + wv)
                        new_s.append(acc[ND + j] + wv * v)
                    return tuple(new_m + new_s)

                flush(lax.fori_loop(0, CH, row_body, zeros, unroll=2))

        for k in range(NBUF - 1):
            @pl.when(k < nch)
            def _prime(k=k):
                pltpu.async_copy(
                    x_hbm.at[b, pl.ds(lo + k * CH, CH), :], bufs[k], sems[k])

        def chunk_body(g, carry):
            for k in range(NBUF):
                @pl.when(g % NBUF == k)
                def _proc(k=k):
                    nk = (k + NBUF - 1) % NBUF
                    process(g, bufs[k], sems[k], bufs[nk], sems[nk])

            return carry

        lax.fori_loop(0, nch, chunk_body, 0, unroll=False)

    ps1, tr1, nb1, f1 = read_params(b1)
    mid1 = f1 + (nb1 - f1 + 1) // 2
    do_range(b1, STEP_LEN * f1, STEP_LEN * mid1, ps1, tr1)
    pltpu.sync_copy(acc_v, shared.at[s, 0])

    ps2, tr2, nb2, f2 = read_params(b2)
    mid2 = f2 + (nb2 - f2 + 1) // 2
    do_range(b2, STEP_LEN * mid2, STEP_LEN * nb2, ps2, tr2)
    pltpu.sync_copy(acc_v, shared.at[NS - 1 - s, 1])

    plsc.subcore_barrier()

    pltpu.sync_copy(shared.at[s, 0], acc_v)
    pltpu.sync_copy(shared.at[s, 1], part_v)
    for j in range(2 * ND):
        part_v[pl.ds(j * L, L)] = (
            acc_v[pl.ds(j * L, L)] + part_v[pl.ds(j * L, L)])
    pltpu.sync_copy(part_v, out_hbm.at[b1])


def _sc_partial(x, current_size, f):
    mesh = plsc.VectorSubcoreMesh(core_axis_name="c", subcore_axis_name="s")
    return pl.kernel(
        _sc_body,
        out_type=jax.ShapeDtypeStruct((B, 2 * D), jnp.float32),
        mesh=mesh,
        compiler_params=pltpu.CompilerParams(needs_layout_passes=False),
        scratch_types=[
            pltpu.VMEM((B,), jnp.float32),
            pltpu.VMEM((B,), jnp.int32),
            pltpu.VMEM((CH, D), jnp.float32),
            pltpu.VMEM((CH, D), jnp.float32),
            pltpu.VMEM((CH, D), jnp.float32),
            pltpu.VMEM((2 * D,), jnp.float32),
            pltpu.VMEM((2 * D,), jnp.float32),
            pltpu.VMEM_SHARED((NS, 2, 2 * D), jnp.float32),
            pltpu.SemaphoreType.DMA,
            pltpu.SemaphoreType.DMA,
            pltpu.SemaphoreType.DMA,
        ],
    )(x, current_size, f)


def _tc_body(tiles_ref, f_ref, psbits_ref, x_ref, out_ref, acc_ref):
    b = pl.program_id(0)
    t = pl.program_id(1)
    psval = lax.bitcast_convert_type(psbits_ref[b], jnp.float32)
    fval = f_ref[b]

    @pl.when(t == 0)
    def _init():
        acc_ref[...] = jnp.zeros_like(acc_ref)

    @pl.when(t * TROWS < fval * STEP_LEN)
    def _acc():
        accm = acc_ref[:, :D]
        accs = acc_ref[:, D:]
        for g in range(TROWS // STEP_LEN):
            tb = t * (TROWS // STEP_LEN) + g
            w = jnp.clip(psval - tb.astype(jnp.float32), 0.0, 1.0)
            w = jnp.where(tb < fval, w, 0.0)
            blk = x_ref[0, pl.ds(g * STEP_LEN, STEP_LEN), :]
            wb = w * blk
            accm = accm + wb
            accs = accs + wb * blk
        acc_ref[:, :D] = accm
        acc_ref[:, D:] = accs

    @pl.when(t == NT - 1)
    def _fold():
        out_ref[...] = jnp.sum(acc_ref[...], axis=0)[None, None, :]


def _tc_partial(x, psbits, f, tc_tiles):
    grid_spec = pltpu.PrefetchScalarGridSpec(
        num_scalar_prefetch=3,
        grid=(B, NT),
        in_specs=[
            pl.BlockSpec(
                (1, TROWS, D),
                lambda b, t, tiles, f, pb: (b, jnp.minimum(t, tiles[b] - 1), 0),
            ),
        ],
        out_specs=pl.BlockSpec((1, 1, 2 * D),
                               lambda b, t, tiles, f, pb: (b, 0, 0)),
        scratch_shapes=[pltpu.VMEM((STEP_LEN, 2 * D), jnp.float32)],
    )
    return pl.pallas_call(
        _tc_body,
        grid_spec=grid_spec,
        out_shape=jax.ShapeDtypeStruct((B, 1, 2 * D), jnp.float32),
        compiler_params=pltpu.CompilerParams(
            dimension_semantics=("arbitrary", "arbitrary")),
    )(tc_tiles, f, psbits, x)


def _fin_body(tc_ref, sc_ref, inv_ref, out_ref):
    tot = tc_ref[...] + sc_ref[...]
    inv = inv_ref[...]
    mean = tot[:, :D] * inv
    var = tot[:, D:] * inv - mean * mean
    out_ref[...] = jnp.concatenate([mean, var], axis=1)


def _finalize(tc_out, sc_out, inv):
    return pl.pallas_call(
        _fin_body,
        out_shape=jax.ShapeDtypeStruct((B, 2 * D), jnp.float32),
    )(tc_out, sc_out, inv)


@jax.jit
def _run(x, current_size):
    ps = jnp.clip(current_size + DEFAULT_SIZE, MIN_SIZE, MAX_SIZE)
    trunc = ps.astype(jnp.int32)
    frac = ps - trunc.astype(jnp.float32)
    nb = trunc + jnp.where(frac > 0.0, 1, 0)
    f = nb // 2
    tc_tiles = jnp.maximum((f * STEP_LEN + TROWS - 1) // TROWS, 1)
    inv = (1.0 / (float(STEP_LEN) * ps))[:, None]

    sc_out = _sc_partial(x, current_size, f)
    tc_out = _tc_partial(x, lax.bitcast_convert_type(ps, jnp.int32), f,
                         tc_tiles)
    return _finalize(tc_out.reshape(B, 2 * D), sc_out, inv)


def kernel(x, current_size):
    return _run(x, current_size)


# speedup vs baseline: 2.5893x; 1.7220x over previous
import functools

import jax
import jax.numpy as jnp
from jax import lax
from jax.experimental import pallas as pl
from jax.experimental.pallas import tpu as pltpu
from jax.experimental.pallas import tpu_sc as plsc

STEP_LEN = 8
MAX_SIZE = 400.0
MIN_SIZE = 1.0
DEFAULT_SIZE = 10.0

NC = 2
NS = 16
L = 16

B = 32
T = 4096
D = 256
ND = D // L
CH = 64
NBUF = 3


def _pool_params(cs_vec):
    ps = jnp.clip(cs_vec + DEFAULT_SIZE, MIN_SIZE, MAX_SIZE)
    trunc_v = ps.astype(jnp.int32)
    frac = ps - trunc_v.astype(jnp.float32)
    nb = (trunc_v + jnp.where(frac > 0.0, 1, 0))[0]
    return ps, trunc_v[0], nb


def _sc_body(x_hbm, cs_hbm, out_hbm, cs_v, buf0, buf1, buf2, acc_v,
             part_v, shared, sem0, sem1, sem2):
    c = lax.axis_index("c")
    s = lax.axis_index("s")
    b1 = 2 * s + c
    b2 = 2 * (NS - 1 - s) + c

    pltpu.sync_copy(cs_hbm, cs_v)

    zero = jnp.zeros((L,), jnp.float32)
    zeros = tuple(zero for _ in range(2 * ND))

    def read_cs(b):
        return plsc.load_gather(cs_v, [jnp.full((L,), b, jnp.int32)])

    bufs = (buf0, buf1, buf2)
    sems = (sem0, sem1, sem2)

    def do_range(b, lo, hi, ps, trunc_s):
        for j in range(2 * ND):
            acc_v[pl.ds(j * L, L)] = zero
        nch = (hi - lo + CH - 1) // CH

        def flush(acc):
            for j in range(2 * ND):
                acc_v[pl.ds(j * L, L)] = acc_v[pl.ds(j * L, L)] + acc[j]

        def process(g, mybuf, mysem, next_buf, next_sem):
            pltpu.make_async_copy(
                x_hbm.at[b, pl.ds(0, CH), :], mybuf, mysem).wait()

            @pl.when(g + NBUF - 1 < nch)
            def _prefetch():
                pltpu.async_copy(
                    x_hbm.at[b, pl.ds(lo + (g + NBUF - 1) * CH, CH), :],
                    next_buf, next_sem)

            start = lo + g * CH
            tb_last = (start + CH - 1) // STEP_LEN
            fast = jnp.logical_and(start + CH <= hi, tb_last + 1 <= trunc_s)

            @pl.when(fast)
            def _fast():
                def row_body(r, acc):
                    new_m = []
                    new_s = []
                    for j in range(ND):
                        v = mybuf[r, pl.ds(j * L, L)]
                        new_m.append(acc[j] + v)
                        new_s.append(acc[ND + j] + v * v)
                    return tuple(new_m + new_s)

                flush(lax.fori_loop(0, CH, row_body, zeros, unroll=2))

            @pl.when(jnp.logical_not(fast))
            def _slow():
                def row_body(r, acc):
                    t = start + r
                    tb = t // STEP_LEN
                    w = jnp.clip(ps - tb.astype(jnp.float32), 0.0, 1.0)
                    w = w * (t < hi).astype(jnp.float32)
                    new_m = []
                    new_s = []
                    for j in range(ND):
                        v = mybuf[r, pl.ds(j * L, L)]
                        wv = w * v
                        new_m.append(acc[j] + wv)
                        new_s.append(acc[ND + j] + wv * v)
                    return tuple(new_m + new_s)

                flush(lax.fori_loop(0, CH, row_body, zeros, unroll=2))

        for k in range(NBUF - 1):
            @pl.when(k < nch)
            def _prime(k=k):
                pltpu.async_copy(
                    x_hbm.at[b, pl.ds(lo + k * CH, CH), :], bufs[k], sems[k])

        def chunk_body(g, carry):
            for k in range(NBUF):
                @pl.when(g % NBUF == k)
                def _proc(k=k):
                    nk = (k + NBUF - 1) % NBUF
                    process(g, bufs[k], sems[k], bufs[nk], sems[nk])

            return carry

        lax.fori_loop(0, nch, chunk_body, 0, unroll=False)

    ps1, tr1, nb1 = _pool_params(read_cs(b1))
    h1 = STEP_LEN * ((nb1 + 1) // 2)
    do_range(b1, 0, h1, ps1, tr1)
    pltpu.sync_copy(acc_v, shared.at[s, 0])

    ps2, tr2, nb2 = _pool_params(read_cs(b2))
    lo2 = STEP_LEN * ((nb2 + 1) // 2)
    hi2 = STEP_LEN * nb2
    do_range(b2, lo2, hi2, ps2, tr2)
    pltpu.sync_copy(acc_v, shared.at[NS - 1 - s, 1])

    plsc.subcore_barrier()

    pltpu.sync_copy(shared.at[s, 0], acc_v)
    pltpu.sync_copy(shared.at[s, 1], part_v)
    inv = 1.0 / (float(STEP_LEN) * ps1)
    for j in range(ND):
        m = (acc_v[pl.ds(j * L, L)] + part_v[pl.ds(j * L, L)]) * inv
        sq = (acc_v[pl.ds((ND + j) * L, L)] + part_v[pl.ds((ND + j) * L, L)]) * inv
        part_v[pl.ds(j * L, L)] = m
        part_v[pl.ds((ND + j) * L, L)] = sq - m * m
    pltpu.sync_copy(part_v, out_hbm.at[b1])


@jax.jit
def _run(x, current_size):
    mesh = plsc.VectorSubcoreMesh(core_axis_name="c", subcore_axis_name="s")
    return pl.kernel(
        _sc_body,
        out_type=jax.ShapeDtypeStruct((B, 2 * D), jnp.float32),
        mesh=mesh,
        compiler_params=pltpu.CompilerParams(needs_layout_passes=False),
        scratch_types=[
            pltpu.VMEM((B,), jnp.float32),
            pltpu.VMEM((CH, D), jnp.float32),
            pltpu.VMEM((CH, D), jnp.float32),
            pltpu.VMEM((CH, D), jnp.float32),
            pltpu.VMEM((2 * D,), jnp.float32),
            pltpu.VMEM((2 * D,), jnp.float32),
            pltpu.VMEM_SHARED((NS, 2, 2 * D), jnp.float32),
            pltpu.SemaphoreType.DMA,
            pltpu.SemaphoreType.DMA,
            pltpu.SemaphoreType.DMA,
        ],
    )(x, current_size)


def kernel(x, current_size):
    return _run(x, current_size)
